# baseline (speedup 1.0000x reference)
import functools

import jax
import jax.numpy as jnp
from jax import lax
from jax.experimental import pallas as pl
from jax.experimental.pallas import tpu as pltpu

_NCORES = 2


def _round_up(x, m):
    return ((x + m - 1) // m) * m


def _focal_block(x, t, a):
    w = x * (1.0 - (t + t))
    e = jnp.exp(-jnp.abs(w))
    one = 1.0 + e
    inv = 1.0 / one
    q = jnp.where(w >= 0.0, inv, e * inv)
    bce = jnp.maximum(w, 0.0) + jnp.log(one)
    return (a * bce) * (q * q)


def _sum_kernel(steps, tile_b, tile_c, rows, need_mask,
                x_ref, t_ref, a_ref, o_ref):
    i = pl.program_id(0)
    j = pl.program_id(1)

    @pl.when(j == 0)
    def _init():
        o_ref[...] = jnp.zeros_like(o_ref)

    x = x_ref[...].astype(jnp.float32)
    t = t_ref[...].astype(jnp.float32)
    a = a_ref[...].astype(jnp.float32)
    loss = _focal_block(x, t, a)
    if need_mask:
        r = ((i * steps + j) * tile_b
             + lax.broadcasted_iota(jnp.int32, (tile_b, tile_c), 0))
        loss = jnp.where(r < rows, loss, 0.0)
    o_ref[...] += jnp.sum(loss).reshape(1, 1, 1)


def kernel(inputs, targets, alpha):
    inputs = jnp.asarray(inputs)
    targets = jnp.asarray(targets)
    B, C = inputs.shape
    alpha_row = jnp.asarray(alpha, jnp.float32).reshape(1, C)

    Cp = _round_up(C, 128)
    per_core = -(-B // _NCORES)
    tile_b = min(2048, _round_up(per_core, 8))
    steps = -(-per_core // tile_b)
    Bp = _NCORES * steps * tile_b
    need_mask = Bp != B

    x2, t2 = inputs, targets
    if Bp != B or Cp != C:
        x2 = jnp.pad(x2, ((0, Bp - B), (0, Cp - C)))
        t2 = jnp.pad(t2, ((0, Bp - B), (0, Cp - C)))
        alpha_row = jnp.pad(alpha_row, ((0, 0), (0, Cp - C)))

    grid = (_NCORES, steps)
    tile_bytes = tile_b * Cp * 4
    vmem_limit = max(int(tile_bytes * 2 * 2 * 1.25) + (2 << 20),
                     16 * 1024 * 1024)

    partials = pl.pallas_call(
        functools.partial(_sum_kernel, steps, tile_b, Cp, B, need_mask),
        out_shape=jax.ShapeDtypeStruct((_NCORES, 1, 1), jnp.float32),
        grid=grid,
        in_specs=[
            pl.BlockSpec((tile_b, Cp), lambda i, j: (i * steps + j, 0)),
            pl.BlockSpec((tile_b, Cp), lambda i, j: (i * steps + j, 0)),
            pl.BlockSpec((1, Cp), lambda i, j: (0, 0)),
        ],
        out_specs=pl.BlockSpec((1, 1, 1), lambda i, j: (i, 0, 0)),
        compiler_params=pltpu.CompilerParams(
            dimension_semantics=("parallel", "arbitrary"),
            vmem_limit_bytes=vmem_limit),
    )(x2, t2, alpha_row)

    return jnp.sum(partials) / jnp.float32(B * C)


# speedup vs baseline: 1.0855x; 1.0855x over previous
import functools

import jax
import jax.numpy as jnp
from jax import lax
from jax.experimental import pallas as pl
from jax.experimental.pallas import tpu as pltpu

_NCORES = 2


def _round_up(x, m):
    return ((x + m - 1) // m) * m


def _focal_block(x, t, a):
    w = x * (1.0 - (t + t))
    e = jnp.exp(-jnp.abs(w))
    one = 1.0 + e
    inv = 1.0 / one
    q = jnp.where(w >= 0.0, inv, e * inv)
    bce = jnp.maximum(w, 0.0) + jnp.log(one)
    return (a * bce) * (q * q)


def _sum_kernel(steps, tile_b, tile_c, rows, need_mask,
                x_ref, t_ref, a_ref, o_ref):
    i = pl.program_id(0)
    j = pl.program_id(1)

    @pl.when(j == 0)
    def _init():
        o_ref[...] = jnp.zeros_like(o_ref)

    x = x_ref[...].astype(jnp.float32)
    t = t_ref[...].astype(jnp.float32)
    a = a_ref[...].astype(jnp.float32)
    loss = _focal_block(x, t, a)
    if need_mask:
        r = ((i * steps + j) * tile_b
             + lax.broadcasted_iota(jnp.int32, (tile_b, tile_c), 0))
        loss = jnp.where(r < rows, loss, 0.0)
    o_ref[...] += jnp.sum(loss).reshape(1, 1, 1)


def kernel(inputs, targets, alpha):
    inputs = jnp.asarray(inputs)
    targets = jnp.asarray(targets)
    B, C = inputs.shape
    alpha_row = jnp.asarray(alpha, jnp.float32).reshape(1, C)

    Cp = _round_up(C, 128)
    per_core = -(-B // _NCORES)
    tile_b = min(4096, _round_up(per_core, 8))
    steps = -(-per_core // tile_b)
    Bp = _NCORES * steps * tile_b
    need_mask = Bp != B

    x2, t2 = inputs, targets
    if Bp != B or Cp != C:
        x2 = jnp.pad(x2, ((0, Bp - B), (0, Cp - C)))
        t2 = jnp.pad(t2, ((0, Bp - B), (0, Cp - C)))
        alpha_row = jnp.pad(alpha_row, ((0, 0), (0, Cp - C)))

    grid = (_NCORES, steps)
    tile_bytes = tile_b * Cp * 4
    vmem_limit = max(int(tile_bytes * 2 * 2 * 1.25) + (2 << 20),
                     16 * 1024 * 1024)

    partials = pl.pallas_call(
        functools.partial(_sum_kernel, steps, tile_b, Cp, B, need_mask),
        out_shape=jax.ShapeDtypeStruct((_NCORES, 1, 1), jnp.float32),
        grid=grid,
        in_specs=[
            pl.BlockSpec((tile_b, Cp), lambda i, j: (i * steps + j, 0)),
            pl.BlockSpec((tile_b, Cp), lambda i, j: (i * steps + j, 0)),
            pl.BlockSpec((1, Cp), lambda i, j: (0, 0)),
        ],
        out_specs=pl.BlockSpec((1, 1, 1), lambda i, j: (i, 0, 0)),
        compiler_params=pltpu.CompilerParams(
            dimension_semantics=("parallel", "arbitrary"),
            vmem_limit_bytes=vmem_limit),
    )(x2, t2, alpha_row)

    return jnp.sum(partials) / jnp.float32(B * C)


# speedup vs baseline: 1.0885x; 1.0028x over previous
import functools

import jax
import jax.numpy as jnp
from jax import lax
from jax.experimental import pallas as pl
from jax.experimental.pallas import tpu as pltpu

_NCORES = 1


def _round_up(x, m):
    return ((x + m - 1) // m) * m


def _focal_block(x, t, a):
    w = x * (1.0 - (t + t))
    e = jnp.exp(-jnp.abs(w))
    one = 1.0 + e
    inv = 1.0 / one
    q = jnp.where(w >= 0.0, inv, e * inv)
    bce = jnp.maximum(w, 0.0) + jnp.log(one)
    return (a * bce) * (q * q)


def _sum_kernel(steps, tile_b, tile_c, rows, need_mask,
                x_ref, t_ref, a_ref, o_ref):
    i = pl.program_id(0)
    j = pl.program_id(1)

    @pl.when(j == 0)
    def _init():
        o_ref[...] = jnp.zeros_like(o_ref)

    x = x_ref[...].astype(jnp.float32)
    t = t_ref[...].astype(jnp.float32)
    a = a_ref[...].astype(jnp.float32)
    loss = _focal_block(x, t, a)
    if need_mask:
        r = ((i * steps + j) * tile_b
             + lax.broadcasted_iota(jnp.int32, (tile_b, tile_c), 0))
        loss = jnp.where(r < rows, loss, 0.0)
    o_ref[...] += jnp.sum(loss).reshape(1, 1, 1)


def kernel(inputs, targets, alpha):
    inputs = jnp.asarray(inputs)
    targets = jnp.asarray(targets)
    B, C = inputs.shape
    alpha_row = jnp.asarray(alpha, jnp.float32).reshape(1, C)

    Cp = _round_up(C, 128)
    per_core = -(-B // _NCORES)
    tile_b = min(4096, _round_up(per_core, 8))
    steps = -(-per_core // tile_b)
    Bp = _NCORES * steps * tile_b
    need_mask = Bp != B

    x2, t2 = inputs, targets
    if Bp != B or Cp != C:
        x2 = jnp.pad(x2, ((0, Bp - B), (0, Cp - C)))
        t2 = jnp.pad(t2, ((0, Bp - B), (0, Cp - C)))
        alpha_row = jnp.pad(alpha_row, ((0, 0), (0, Cp - C)))

    grid = (_NCORES, steps)
    tile_bytes = tile_b * Cp * 4
    vmem_limit = max(int(tile_bytes * 2 * 2 * 1.25) + (2 << 20),
                     16 * 1024 * 1024)

    partials = pl.pallas_call(
        functools.partial(_sum_kernel, steps, tile_b, Cp, B, need_mask),
        out_shape=jax.ShapeDtypeStruct((_NCORES, 1, 1), jnp.float32),
        grid=grid,
        in_specs=[
            pl.BlockSpec((tile_b, Cp), lambda i, j: (i * steps + j, 0)),
            pl.BlockSpec((tile_b, Cp), lambda i, j: (i * steps + j, 0)),
            pl.BlockSpec((1, Cp), lambda i, j: (0, 0)),
        ],
        out_specs=pl.BlockSpec((1, 1, 1), lambda i, j: (i, 0, 0)),
        compiler_params=pltpu.CompilerParams(
            dimension_semantics=("parallel", "arbitrary"),
            vmem_limit_bytes=vmem_limit),
    )(x2, t2, alpha_row)

    return jnp.sum(partials) / jnp.float32(B * C)


# speedup vs baseline: 1.3421x; 1.2330x over previous
import functools

import jax
import jax.numpy as jnp
from jax import lax
from jax.experimental import pallas as pl
from jax.experimental.pallas import tpu as pltpu

_NCORES = 1


def _round_up(x, m):
    return ((x + m - 1) // m) * m


def _focal_block(x, t, a):
    w = x * (1.0 - (t + t))
    e = jnp.exp(-jnp.abs(w))
    one = 1.0 + e
    inv = 1.0 / one
    q = jnp.where(w >= 0.0, inv, e * inv)
    bce = jnp.maximum(w, 0.0) + jnp.log(one)
    return (a * bce) * (q * q)


def _sum_kernel(steps, tile_b, tile_c, rows, need_mask,
                x_ref, t_ref, a_ref, o_ref):
    i = pl.program_id(0)
    j = pl.program_id(1)

    @pl.when(j == 0)
    def _init():
        o_ref[...] = jnp.zeros_like(o_ref)

    x = x_ref[...].astype(jnp.float32)
    t = t_ref[...].astype(jnp.float32)
    a = a_ref[...].astype(jnp.float32)
    loss = _focal_block(x, t, a)
    if need_mask:
        r = ((i * steps + j) * tile_b
             + lax.broadcasted_iota(jnp.int32, (tile_b, tile_c), 0))
        loss = jnp.where(r < rows, loss, 0.0)
    o_ref[...] += jnp.sum(loss).reshape(1, 1, 1)


def kernel(inputs, targets, alpha):
    inputs = jnp.asarray(inputs)
    targets = jnp.asarray(targets)
    B, C = inputs.shape
    alpha_row = jnp.asarray(alpha, jnp.float32).reshape(1, C)

    Cp = _round_up(C, 128)
    per_core = -(-B // _NCORES)
    tile_b = min(4096, _round_up(per_core, 8))
    steps = -(-per_core // tile_b)
    Bp = _NCORES * steps * tile_b
    need_mask = Bp != B

    x2, t2 = inputs, targets
    if Bp != B or Cp != C:
        x2 = jnp.pad(x2, ((0, Bp - B), (0, Cp - C)))
        t2 = jnp.pad(t2, ((0, Bp - B), (0, Cp - C)))
        alpha_row = jnp.pad(alpha_row, ((0, 0), (0, Cp - C)))

    grid = (_NCORES, steps)
    vmem_limit = 100 * 1024 * 1024

    partials = pl.pallas_call(
        functools.partial(_sum_kernel, steps, tile_b, Cp, B, need_mask),
        out_shape=jax.ShapeDtypeStruct((_NCORES, 1, 1), jnp.float32),
        grid=grid,
        in_specs=[
            pl.BlockSpec((tile_b, Cp), lambda i, j: (i * steps + j, 0)),
            pl.BlockSpec((tile_b, Cp), lambda i, j: (i * steps + j, 0)),
            pl.BlockSpec((1, Cp), lambda i, j: (0, 0)),
        ],
        out_specs=pl.BlockSpec((1, 1, 1), lambda i, j: (i, 0, 0)),
        compiler_params=pltpu.CompilerParams(
            dimension_semantics=("parallel", "arbitrary"),
            vmem_limit_bytes=vmem_limit),
    )(x2, t2, alpha_row)

    return jnp.sum(partials) / jnp.float32(B * C)


# speedup vs baseline: 1.3577x; 1.0116x over previous
import functools

import jax
import jax.numpy as jnp
from jax import lax
from jax.experimental import pallas as pl
from jax.experimental.pallas import tpu as pltpu

_NCORES = 1


def _round_up(x, m):
    return ((x + m - 1) // m) * m


def _focal_block(x, t, a):
    w = x * (1.0 - (t + t))
    e = jnp.exp(-jnp.abs(w))
    one = 1.0 + e
    inv = 1.0 / one
    q = jnp.where(w >= 0.0, inv, e * inv)
    bce = jnp.maximum(w, 0.0) + jnp.log(one)
    return (a * bce) * (q * q)


def _sum_kernel(steps, tile_b, tile_c, rows, need_mask,
                x_ref, t_ref, a_ref, o_ref):
    i = pl.program_id(0)
    j = pl.program_id(1)

    @pl.when(j == 0)
    def _init():
        o_ref[...] = jnp.zeros_like(o_ref)

    x = x_ref[...].astype(jnp.float32)
    t = t_ref[...].astype(jnp.float32)
    a = a_ref[...].astype(jnp.float32)
    loss = _focal_block(x, t, a)
    if need_mask:
        r = ((i * steps + j) * tile_b
             + lax.broadcasted_iota(jnp.int32, (tile_b, tile_c), 0))
        loss = jnp.where(r < rows, loss, 0.0)
    o_ref[...] += jnp.sum(loss).reshape(1, 1, 1)


def kernel(inputs, targets, alpha):
    inputs = jnp.asarray(inputs)
    targets = jnp.asarray(targets)
    B, C = inputs.shape
    alpha_row = jnp.asarray(alpha, jnp.float32).reshape(1, C)

    Cp = _round_up(C, 128)
    per_core = -(-B // _NCORES)
    tile_b = min(8192, _round_up(per_core, 8))
    steps = -(-per_core // tile_b)
    Bp = _NCORES * steps * tile_b
    need_mask = Bp != B

    x2, t2 = inputs, targets
    if Bp != B or Cp != C:
        x2 = jnp.pad(x2, ((0, Bp - B), (0, Cp - C)))
        t2 = jnp.pad(t2, ((0, Bp - B), (0, Cp - C)))
        alpha_row = jnp.pad(alpha_row, ((0, 0), (0, Cp - C)))

    grid = (_NCORES, steps)
    vmem_limit = 100 * 1024 * 1024

    partials = pl.pallas_call(
        functools.partial(_sum_kernel, steps, tile_b, Cp, B, need_mask),
        out_shape=jax.ShapeDtypeStruct((_NCORES, 1, 1), jnp.float32),
        grid=grid,
        in_specs=[
            pl.BlockSpec((tile_b, Cp), lambda i, j: (i * steps + j, 0)),
            pl.BlockSpec((tile_b, Cp), lambda i, j: (i * steps + j, 0)),
            pl.BlockSpec((1, Cp), lambda i, j: (0, 0)),
        ],
        out_specs=pl.BlockSpec((1, 1, 1), lambda i, j: (i, 0, 0)),
        compiler_params=pltpu.CompilerParams(
            dimension_semantics=("parallel", "arbitrary"),
            vmem_limit_bytes=vmem_limit),
    )(x2, t2, alpha_row)

    return jnp.sum(partials) / jnp.float32(B * C)
